# Initial kernel scaffold; baseline (speedup 1.0000x reference)
import jax
import jax.numpy as jnp
from jax.experimental import pallas as pl


def kernel(node_feats, children, W1, b1, W2, b2):
    raise NotImplementedError("write your pallas kernel here")


# speedup vs baseline: 938.4117x; 938.4117x over previous
import functools

import jax
import jax.numpy as jnp
from jax import lax
from jax.experimental import pallas as pl
from jax.experimental.pallas import tpu as pltpu
from jax.experimental.pallas import tpu_sc as plsc

B = 2048
N = 255
NODES = 256
C = 64
KO = 192
L = 768

NC, NS = 2, 16
NW = NC * NS
BPW = B // NW

TOT = NODES * C


_BS1 = 16


def _proj_body(x_ref, w_ref, o_ref):
    for j in range(_BS1):
        o_ref[j] = lax.dot_general(
            x_ref[j], w_ref[...], (((0,), (0,)), ((), ())),
            preferred_element_type=jnp.float32)


def _proj(x, wc):
    return pl.pallas_call(
        _proj_body,
        grid=(B // _BS1,),
        in_specs=[
            pl.BlockSpec((_BS1, C, NODES), lambda i: (i, 0, 0)),
            pl.BlockSpec((C, KO), lambda i: (0, 0)),
        ],
        out_specs=pl.BlockSpec((_BS1, NODES, KO), lambda i: (i, 0, 0)),
        out_shape=jax.ShapeDtypeStruct((B, NODES, KO), jnp.float32),
    )(x, wc)


_BS2 = 16


def _ln_proj_body(g_ref, bias_ref, w_ref, o_ref):
    for j in range(_BS2):
        t = g_ref[j] + bias_ref[...]
        mu = jnp.sum(t) / TOT
        d = t - mu
        s2 = jnp.sum(d * d) + C * mu * mu
        inv = 1.0 / (jnp.sqrt(s2 / (TOT - 1)) + 1e-5)
        xn = jnp.maximum(d * inv, 0.0)
        x0 = jnp.maximum(-mu * inv, 0.0)
        row0 = jnp.broadcast_to(x0, (1, C))
        x = jnp.concatenate([row0, xn], axis=0)
        o_ref[j] = jnp.dot(x, w_ref[...], preferred_element_type=jnp.float32)


def _ln_proj(g, bias, wc):
    return pl.pallas_call(
        _ln_proj_body,
        grid=(B // _BS2,),
        in_specs=[
            pl.BlockSpec((_BS2, N, C), lambda i: (i, 0, 0)),
            pl.BlockSpec((1, C), lambda i: (0, 0)),
            pl.BlockSpec((C, KO), lambda i: (0, 0)),
        ],
        out_specs=pl.BlockSpec((_BS2, NODES, KO), lambda i: (i, 0, 0)),
        out_shape=jax.ShapeDtypeStruct((B, NODES, KO), jnp.float32),
    )(g, bias, wc)


_BS3 = 64


def _ln_pool_body(g_ref, bias_ref, o_ref):
    for j in range(_BS3):
        t = g_ref[j] + bias_ref[...]
        mu = jnp.sum(t) / TOT
        d = t - mu
        s2 = jnp.sum(d * d) + C * mu * mu
        inv = 1.0 / (jnp.sqrt(s2 / (TOT - 1)) + 1e-5)
        xn = jnp.maximum(d * inv, 0.0)
        x0 = jnp.maximum(-mu * inv, 0.0)
        m = jnp.max(xn, axis=0)
        o_ref[j] = jnp.maximum(m, x0)


def _ln_pool(g, bias):
    return pl.pallas_call(
        _ln_pool_body,
        grid=(B // _BS3,),
        in_specs=[
            pl.BlockSpec((_BS3, N, C), lambda i: (i, 0, 0)),
            pl.BlockSpec((1, C), lambda i: (0, 0)),
        ],
        out_specs=pl.BlockSpec((_BS3, C), lambda i: (i, 0)),
        out_shape=jax.ShapeDtypeStruct((B, C), jnp.float32),
    )(g, bias)


_mesh = plsc.VectorSubcoreMesh(core_axis_name="c", subcore_axis_name="s")


@functools.partial(
    pl.kernel,
    out_type=jax.ShapeDtypeStruct((B, N, C), jnp.float32),
    mesh=_mesh,
    scratch_types=[
        pltpu.VMEM((6, 128), jnp.int32),
        pltpu.VMEM((L, C), jnp.float32),
        pltpu.SemaphoreType.DMA,
    ],
    compiler_params=pltpu.CompilerParams(use_tc_tiling_on_sc=False),
)
def _gather_sum(table_hbm, idx_hbm, out_hbm, idx_v, rows_v, sem):
    wid = lax.axis_index("s") * NC + lax.axis_index("c")

    def body(i, carry):
        b = wid * BPW + i
        pltpu.sync_copy(idx_hbm.at[b], idx_v)
        copies = [
            pltpu.async_copy(
                table_hbm.at[idx_v.at[r]],
                rows_v.at[pl.ds(r * 128, 128)], sem)
            for r in range(6)
        ]
        for cp in copies:
            cp.wait()

        def add_body(n, c2):
            for j in range(C // 16):
                sl = pl.ds(j * 16, 16)
                rows_v[n, sl] = (rows_v[n, sl]
                                 + rows_v[n + 256, sl]
                                 + rows_v[n + 512, sl])
            return c2

        lax.fori_loop(0, N, add_body, 0)
        pltpu.sync_copy(rows_v.at[pl.ds(0, N)], out_hbm.at[b])
        return carry

    lax.fori_loop(0, BPW, body, 0)


def kernel(node_feats, children, W1, b1, W2, b2):
    wc1 = jnp.transpose(W1, (1, 2, 0)).reshape(C, KO)
    wc2 = jnp.transpose(W2, (1, 2, 0)).reshape(C, KO)

    ch = children[:, :, 0].reshape(B, N, 3)
    chk = jnp.pad(jnp.transpose(ch, (0, 2, 1)), ((0, 0), (0, 0), (0, 1)))
    kk = jnp.arange(3, dtype=jnp.int32).reshape(1, 3, 1)
    boff = (jnp.arange(B, dtype=jnp.int32) * L).reshape(B, 1, 1)
    idxg = (chk * 3 + kk + boff).reshape(B, 6, 128)

    h1 = _proj(node_feats, wc1)
    g1 = _gather_sum(h1.reshape(B * NODES * 3, C), idxg)
    h2 = _ln_proj(g1, b1.reshape(1, C), wc2)
    g2 = _gather_sum(h2.reshape(B * NODES * 3, C), idxg)
    return _ln_pool(g2, b2.reshape(1, C))


# speedup vs baseline: 1826.6179x; 1.9465x over previous
import functools

import jax
import jax.numpy as jnp
from jax import lax
from jax.experimental import pallas as pl
from jax.experimental.pallas import tpu as pltpu
from jax.experimental.pallas import tpu_sc as plsc

B = 2048
N = 255
NODES = 256
C = 64
W128 = 128
L = 768
TBL = B * NODES

NC, NS = 2, 16
NW = NC * NS
BPW = B // NW

TOT = NODES * C


_BS1 = 16


def _proj_body(x_ref, w01_ref, w2_ref, o_ref):
    xt = jnp.transpose(x_ref[...], (0, 2, 1)).reshape(_BS1 * NODES, C)
    h01 = jnp.dot(xt, w01_ref[...], preferred_element_type=jnp.float32)
    h2 = jnp.dot(xt, w2_ref[...], preferred_element_type=jnp.float32)
    o_ref[0] = h01.reshape(_BS1, NODES, W128)
    o_ref[1] = h2.reshape(_BS1, NODES, W128)


def _proj(x, w01, w2):
    return pl.pallas_call(
        _proj_body,
        grid=(B // _BS1,),
        in_specs=[
            pl.BlockSpec((_BS1, C, NODES), lambda i: (i, 0, 0)),
            pl.BlockSpec((C, W128), lambda i: (0, 0)),
            pl.BlockSpec((C, W128), lambda i: (0, 0)),
        ],
        out_specs=pl.BlockSpec((2, _BS1, NODES, W128), lambda i: (0, i, 0, 0)),
        out_shape=jax.ShapeDtypeStruct((2, B, NODES, W128), jnp.float32),
    )(x, w01, w2)


_BS2 = 16


def _ln_stats(g_blk, bias_ref, bs):
    t = g_blk[:, :N, :] + bias_ref[...]
    mu = (jnp.sum(jnp.where(_lane_mask(bs), t, 0.0), axis=(1, 2))
          / TOT)[:, None, None]
    dm = jnp.where(_lane_mask(bs), t - mu, 0.0)
    s2 = jnp.sum(dm * dm, axis=(1, 2))[:, None, None] + C * mu * mu
    inv = 1.0 / (jnp.sqrt(s2 / (TOT - 1)) + 1e-5)
    xn = jnp.maximum(dm * inv, 0.0)
    x0 = jnp.maximum(-mu * inv, 0.0)
    return xn, x0


def _lane_mask(bs):
    return lax.broadcasted_iota(jnp.int32, (bs, N, W128), 2) < C


def _ln_proj_body(g_ref, bias_ref, w01_ref, w2_ref, o_ref):
    xn, x0 = _ln_stats(g_ref[...], bias_ref, _BS2)
    x = jnp.concatenate(
        [jnp.broadcast_to(x0, (_BS2, 1, W128)), xn], axis=1)
    xf = x.reshape(_BS2 * NODES, W128)
    h01 = jnp.dot(xf, w01_ref[...], preferred_element_type=jnp.float32)
    h2 = jnp.dot(xf, w2_ref[...], preferred_element_type=jnp.float32)
    o_ref[0] = h01.reshape(_BS2, NODES, W128)
    o_ref[1] = h2.reshape(_BS2, NODES, W128)


def _ln_proj(g, bias, w01, w2):
    return pl.pallas_call(
        _ln_proj_body,
        grid=(B // _BS2,),
        in_specs=[
            pl.BlockSpec((_BS2, NODES, W128), lambda i: (i, 0, 0)),
            pl.BlockSpec((1, W128), lambda i: (0, 0)),
            pl.BlockSpec((W128, W128), lambda i: (0, 0)),
            pl.BlockSpec((W128, W128), lambda i: (0, 0)),
        ],
        out_specs=pl.BlockSpec((2, _BS2, NODES, W128), lambda i: (0, i, 0, 0)),
        out_shape=jax.ShapeDtypeStruct((2, B, NODES, W128), jnp.float32),
    )(g, bias, w01, w2)


_BS3 = 64


def _ln_pool_body(g_ref, bias_ref, o_ref):
    xn, x0 = _ln_stats(g_ref[...], bias_ref, _BS3)
    m = jnp.maximum(jnp.max(xn, axis=1), x0[:, 0, :])
    o_ref[...] = m[:, :C]


def _ln_pool(g, bias):
    return pl.pallas_call(
        _ln_pool_body,
        grid=(B // _BS3,),
        in_specs=[
            pl.BlockSpec((_BS3, NODES, W128), lambda i: (i, 0, 0)),
            pl.BlockSpec((1, W128), lambda i: (0, 0)),
        ],
        out_specs=pl.BlockSpec((_BS3, C), lambda i: (i, 0)),
        out_shape=jax.ShapeDtypeStruct((B, C), jnp.float32),
    )(g, bias)


_mesh = plsc.VectorSubcoreMesh(core_axis_name="c", subcore_axis_name="s")


@functools.partial(
    pl.kernel,
    out_type=jax.ShapeDtypeStruct((B, NODES, W128), jnp.float32),
    mesh=_mesh,
    scratch_types=[
        pltpu.VMEM((6, 128), jnp.int32),
        pltpu.VMEM((L, W128), jnp.float32),
        pltpu.SemaphoreType.DMA,
    ],
    compiler_params=pltpu.CompilerParams(use_tc_tiling_on_sc=False),
)
def _gather_sum(table_hbm, idx_hbm, out_hbm, idx_v, rows_v, sem):
    wid = lax.axis_index("s") * NC + lax.axis_index("c")

    def body(i, carry):
        b = wid * BPW + i
        pltpu.sync_copy(idx_hbm.at[pl.ds(b * 6, 6)], idx_v)
        copies = [
            pltpu.async_copy(
                table_hbm.at[idx_v.at[r]],
                rows_v.at[pl.ds(r * 128, 128)], sem)
            for r in range(6)
        ]
        for cp in copies:
            cp.wait()

        def add_body(n, c2):
            for j in range(C // 16):
                sl = pl.ds(j * 16, 16)
                sl1 = pl.ds(C + j * 16, 16)
                rows_v[n, sl] = (rows_v[n, sl]
                                 + rows_v[n + 256, sl1]
                                 + rows_v[n + 512, sl])
            return c2

        lax.fori_loop(0, N, add_body, 0)
        pltpu.sync_copy(rows_v.at[pl.ds(0, NODES)], out_hbm.at[b])
        return carry

    lax.fori_loop(0, BPW, body, 0)


def _pack_w(Wt):
    w = jnp.transpose(Wt, (2, 1, 0))
    z = jnp.zeros((3, C, C), w.dtype)
    w01 = jnp.concatenate([w[0], w[1]], axis=1)
    w2 = jnp.concatenate([w[2], z[0]], axis=1)
    pad = jnp.zeros((C, W128), w.dtype)
    w01f = jnp.concatenate([w01, pad], axis=0)
    w2f = jnp.concatenate([w2, pad], axis=0)
    return w01, w2, w01f, w2f


def kernel(node_feats, children, W1, b1, W2, b2):
    w01a, w2a, _, _ = _pack_w(W1)
    _, _, w01b, w2b = _pack_w(W2)

    ch = children[:, :, 0].reshape(B, N, 3)
    chk = jnp.pad(jnp.transpose(ch, (0, 2, 1)), ((0, 0), (0, 0), (0, 1)))
    kk = jnp.array([0, 0, TBL], jnp.int32).reshape(1, 3, 1)
    boff = (jnp.arange(B, dtype=jnp.int32) * NODES).reshape(B, 1, 1)
    idxg = (chk + kk + boff).reshape(B * 6, 128)

    b1p = jnp.pad(b1.reshape(1, C), ((0, 0), (0, C)))
    b2p = jnp.pad(b2.reshape(1, C), ((0, 0), (0, C)))

    h1 = _proj(node_feats, w01a, w2a)
    g1 = _gather_sum(h1.reshape(2 * TBL, W128), idxg)
    h2 = _ln_proj(g1, b1p, w01b, w2b)
    g2 = _gather_sum(h2.reshape(2 * TBL, W128), idxg)
    return _ln_pool(g2, b2p)


# speedup vs baseline: 2271.8399x; 1.2437x over previous
import functools

import jax
import jax.numpy as jnp
from jax import lax
from jax.experimental import pallas as pl
from jax.experimental.pallas import tpu as pltpu
from jax.experimental.pallas import tpu_sc as plsc

B = 2048
N = 255
NODES = 256
C = 64
W128 = 128
L = 768
TBL = B * NODES

NC, NS = 2, 16
NW = NC * NS
BPW = B // NW

TOT = NODES * C


_BS1 = 16


def _proj_body(x_ref, w01_ref, w2_ref, o_ref):
    xt = jnp.transpose(x_ref[...], (0, 2, 1)).reshape(_BS1 * NODES, C)
    h01 = jnp.dot(xt, w01_ref[...], preferred_element_type=jnp.float32)
    h2 = jnp.dot(xt, w2_ref[...], preferred_element_type=jnp.float32)
    o_ref[0] = h01.reshape(_BS1, NODES, W128)
    o_ref[1] = h2.reshape(_BS1, NODES, W128)


def _proj(x, w01, w2):
    return pl.pallas_call(
        _proj_body,
        grid=(B // _BS1,),
        in_specs=[
            pl.BlockSpec((_BS1, C, NODES), lambda i: (i, 0, 0)),
            pl.BlockSpec((C, W128), lambda i: (0, 0)),
            pl.BlockSpec((C, W128), lambda i: (0, 0)),
        ],
        out_specs=pl.BlockSpec((2, _BS1, NODES, W128), lambda i: (0, i, 0, 0)),
        out_shape=jax.ShapeDtypeStruct((2, B, NODES, W128), jnp.float32),
    )(x, w01, w2)


_BS2 = 16


def _ln_stats(g_blk, bias_ref, bs):
    t = g_blk[:, :N, :] + bias_ref[...]
    mu = (jnp.sum(jnp.where(_lane_mask(bs), t, 0.0), axis=(1, 2))
          / TOT)[:, None, None]
    dm = jnp.where(_lane_mask(bs), t - mu, 0.0)
    s2 = jnp.sum(dm * dm, axis=(1, 2))[:, None, None] + C * mu * mu
    inv = 1.0 / (jnp.sqrt(s2 / (TOT - 1)) + 1e-5)
    xn = jnp.maximum(dm * inv, 0.0)
    x0 = jnp.maximum(-mu * inv, 0.0)
    return xn, x0


def _lane_mask(bs):
    return lax.broadcasted_iota(jnp.int32, (bs, N, W128), 2) < C


def _ln_proj_body(g_ref, bias_ref, w01_ref, w2_ref, o_ref):
    xn, x0 = _ln_stats(g_ref[...], bias_ref, _BS2)
    x = jnp.concatenate(
        [jnp.broadcast_to(x0, (_BS2, 1, W128)), xn], axis=1)
    xf = x.reshape(_BS2 * NODES, W128)
    h01 = jnp.dot(xf, w01_ref[...], preferred_element_type=jnp.float32)
    h2 = jnp.dot(xf, w2_ref[...], preferred_element_type=jnp.float32)
    o_ref[0] = h01.reshape(_BS2, NODES, W128)
    o_ref[1] = h2.reshape(_BS2, NODES, W128)


def _ln_proj(g, bias, w01, w2):
    return pl.pallas_call(
        _ln_proj_body,
        grid=(B // _BS2,),
        in_specs=[
            pl.BlockSpec((_BS2, NODES, W128), lambda i: (i, 0, 0)),
            pl.BlockSpec((1, W128), lambda i: (0, 0)),
            pl.BlockSpec((W128, W128), lambda i: (0, 0)),
            pl.BlockSpec((W128, W128), lambda i: (0, 0)),
        ],
        out_specs=pl.BlockSpec((2, _BS2, NODES, W128), lambda i: (0, i, 0, 0)),
        out_shape=jax.ShapeDtypeStruct((2, B, NODES, W128), jnp.float32),
    )(g, bias, w01, w2)


_BS3 = 64


def _ln_pool_body(g_ref, bias_ref, o_ref):
    xn, x0 = _ln_stats(g_ref[...], bias_ref, _BS3)
    m = jnp.maximum(jnp.max(xn, axis=1), x0[:, 0, :])
    o_ref[...] = m[:, :C]


def _ln_pool(g, bias):
    return pl.pallas_call(
        _ln_pool_body,
        grid=(B // _BS3,),
        in_specs=[
            pl.BlockSpec((_BS3, NODES, W128), lambda i: (i, 0, 0)),
            pl.BlockSpec((1, W128), lambda i: (0, 0)),
        ],
        out_specs=pl.BlockSpec((_BS3, C), lambda i: (i, 0)),
        out_shape=jax.ShapeDtypeStruct((B, C), jnp.float32),
    )(g, bias)


_mesh = plsc.VectorSubcoreMesh(core_axis_name="c", subcore_axis_name="s")


@functools.partial(
    pl.kernel,
    out_type=jax.ShapeDtypeStruct((B, NODES, W128), jnp.float32),
    mesh=_mesh,
    scratch_types=[
        pltpu.VMEM((2, 6, 128), jnp.int32),
        pltpu.VMEM((2, 384, W128), jnp.float32),
        pltpu.VMEM((128, W128), jnp.float32),
        pltpu.SemaphoreType.DMA,
        pltpu.SemaphoreType.DMA,
        pltpu.SemaphoreType.DMA,
        pltpu.SemaphoreType.DMA,
    ],
    compiler_params=pltpu.CompilerParams(use_tc_tiling_on_sc=False),
)
def _gather_sum(table_hbm, idx_hbm, out_hbm, idx_v, buf_v, ost_v,
                g0, g1, isem, osem):
    wid = lax.axis_index("s") * NC + lax.axis_index("c")
    b0 = wid * BPW
    gsem = (g0, g1)

    def fire_idx(i):
        pltpu.async_copy(idx_hbm.at[pl.ds((b0 + i) * 6, 6)],
                         idx_v.at[i % 2], isem)

    def fire_half(i, q):
        for s in range(3):
            pltpu.async_copy(
                table_hbm.at[idx_v.at[i % 2, 2 * s + q]],
                buf_v.at[q, pl.ds(s * 128, 128)], gsem[q])

    def drain_half(q):
        pltpu.make_async_copy(table_hbm.at[pl.ds(0, 384)],
                              buf_v.at[q], gsem[q]).wait()

    def drain_idx(i):
        pltpu.make_async_copy(idx_hbm.at[pl.ds(0, 6)],
                              idx_v.at[i % 2], isem).wait()

    def drain_out():
        pltpu.make_async_copy(out_hbm.at[0, pl.ds(0, 128)],
                              ost_v, osem).wait()

    def add_half(q):
        def ab(m, c2):
            for j in range(C // 16):
                sl = pl.ds(j * 16, 16)
                sl1 = pl.ds(C + j * 16, 16)
                ost_v[m, sl] = (buf_v[q, m, sl]
                                + buf_v[q, 128 + m, sl1]
                                + buf_v[q, 256 + m, sl])
            return c2
        lax.fori_loop(0, 128, ab, 0)

    pltpu.sync_copy(idx_hbm.at[pl.ds(b0 * 6, 6)], idx_v.at[0])
    fire_half(0, 0)
    fire_half(0, 1)

    @pl.when(BPW > 1)
    def _():
        fire_idx(1)

    def body(i, carry):
        b = b0 + i
        drain_half(0)

        @pl.when(i > 0)
        def _():
            drain_out()
        add_half(0)
        pltpu.async_copy(ost_v, out_hbm.at[b, pl.ds(0, 128)], osem)

        @pl.when(i < BPW - 1)
        def _():
            drain_idx(i + 1)
            fire_half(i + 1, 0)

        @pl.when(i < BPW - 2)
        def _():
            fire_idx(i + 2)

        drain_half(1)
        drain_out()
        add_half(1)
        pltpu.async_copy(ost_v, out_hbm.at[b, pl.ds(128, 128)], osem)

        @pl.when(i < BPW - 1)
        def _():
            fire_half(i + 1, 1)
        return carry

    lax.fori_loop(0, BPW, body, 0)
    drain_out()


def _pack_w(Wt):
    w = jnp.transpose(Wt, (2, 1, 0))
    z = jnp.zeros((3, C, C), w.dtype)
    w01 = jnp.concatenate([w[0], w[1]], axis=1)
    w2 = jnp.concatenate([w[2], z[0]], axis=1)
    pad = jnp.zeros((C, W128), w.dtype)
    w01f = jnp.concatenate([w01, pad], axis=0)
    w2f = jnp.concatenate([w2, pad], axis=0)
    return w01, w2, w01f, w2f


def kernel(node_feats, children, W1, b1, W2, b2):
    w01a, w2a, _, _ = _pack_w(W1)
    _, _, w01b, w2b = _pack_w(W2)

    ch = children[:, :, 0].reshape(B, N, 3)
    chk = jnp.pad(jnp.transpose(ch, (0, 2, 1)), ((0, 0), (0, 0), (0, 1)))
    kk = jnp.array([0, 0, TBL], jnp.int32).reshape(1, 3, 1)
    boff = (jnp.arange(B, dtype=jnp.int32) * NODES).reshape(B, 1, 1)
    idxg = (chk + kk + boff).reshape(B * 6, 128)

    b1p = jnp.pad(b1.reshape(1, C), ((0, 0), (0, C)))
    b2p = jnp.pad(b2.reshape(1, C), ((0, 0), (0, C)))

    h1 = _proj(node_feats, w01a, w2a)
    g1 = _gather_sum(h1.reshape(2 * TBL, W128), idxg)
    h2 = _ln_proj(g1, b1p, w01b, w2b)
    g2 = _gather_sum(h2.reshape(2 * TBL, W128), idxg)
    return _ln_pool(g2, b2p)
